# baseline (speedup 1.0000x reference)
import functools

import jax
import jax.numpy as jnp
from jax import lax
from jax.experimental import pallas as pl
from jax.experimental.pallas import tpu as pltpu
from jax.experimental.pallas import tpu_sc as plsc

N = 100000
D = 128
DH = 256
M = 50000

NC = 2
NS = 16
L = 16
NW = NC * NS
PER_W = 1568
M_PAD = NW * PER_W
CK = 112
NCK = PER_W // CK
GPC = CK // L
RB = 3
NCH = 8


def _rsqrt16(x):
    i = plsc.bitcast(x, jnp.int32)
    y = plsc.bitcast(jnp.int32(0x5F3759DF) - (i >> 1), jnp.float32)
    for _ in range(3):
        y = y * (1.5 - 0.5 * x * y * y)
    return y


@functools.lru_cache(maxsize=1)
def _make_sc_loss():
    mesh = plsc.VectorSubcoreMesh(
        core_axis_name="c", subcore_axis_name="s", num_cores=NC, num_subcores=NS
    )

    @functools.partial(
        pl.kernel,
        out_type=jax.ShapeDtypeStruct((NW, L), jnp.float32),
        mesh=mesh,
        compiler_params=pltpu.CompilerParams(needs_layout_passes=False),
        scratch_types=[
            pltpu.VMEM((NCK, CK), jnp.int32),
            pltpu.VMEM((DH,), jnp.float32),
            pltpu.VMEM((DH, D), jnp.float32),
            pltpu.VMEM((D,), jnp.float32),
            pltpu.VMEM((D,), jnp.float32),
            pltpu.VMEM((RB, CK, D), jnp.float32),
            pltpu.VMEM((L,), jnp.float32),
            pltpu.SemaphoreType.DMA,
            pltpu.SemaphoreType.DMA,
            pltpu.SemaphoreType.DMA,
        ],
    )
    def _sc_loss(dstx_hbm, idx_hbm, b1_hbm, w2_hbm, b2_hbm, out_hbm,
                 idx_v, b1_v, w2_v, b2_v, vn_v, rows, acc_v,
                 sem0, sem1, sem2):
        wid = lax.axis_index("s") * NC + lax.axis_index("c")
        base = wid * PER_W
        pltpu.sync_copy(idx_hbm.at[wid], idx_v)
        pltpu.sync_copy(b1_hbm, b1_v)
        pltpu.sync_copy(w2_hbm, w2_v)
        pltpu.sync_copy(b2_hbm, b2_v)

        sems = (sem0, sem1, sem2)

        def fire(k):
            return pltpu.async_copy(
                dstx_hbm.at[idx_v.at[k]], rows.at[k % RB], sems[k % RB])

        pend = [fire(k) for k in range(RB - 1)]

        def vbody(k, vacc):
            bk = jnp.maximum(
                plsc.load_gather(b1_v, [jnp.full((L,), k, jnp.int32)]), 0.0)
            return tuple(
                vacc[j] + bk * w2_v[k, pl.ds(j * L, L)] for j in range(NCH))

        vacc = lax.fori_loop(
            0, DH, vbody, tuple(jnp.zeros((L,), jnp.float32) for _ in range(NCH)),
            unroll=2)
        vch = [vacc[j] + b2_v[pl.ds(j * L, L)] for j in range(NCH)]
        nsq_vec = vch[0] * vch[0]
        for j in range(1, NCH):
            nsq_vec = nsq_vec + vch[j] * vch[j]
        nsq = jnp.full((L,), lax.reduce_sum(nsq_vec, axes=(0,)))
        vnorm = nsq * _rsqrt16(jnp.maximum(nsq, 1e-30))
        inv = 1.0 / jnp.maximum(vnorm, 1e-8)
        for j in range(NCH):
            vn_v[pl.ds(j * L, L)] = vch[j] * inv

        iota = lax.iota(jnp.int32, L)
        rowsel = [s * L + iota for s in range(GPC)]

        acc = jnp.zeros((L,), jnp.float32)
        for k in range(NCK):
            if k + RB - 1 < NCK:
                pend.append(fire(k + RB - 1))
            pend.pop(0).wait()
            cur = rows.at[k % RB]

            def dbody(d, carry):
                dots, qs = carry
                cold = jnp.full((L,), d, jnp.int32)
                vnd = plsc.load_gather(vn_v, [cold])
                nd = []
                nq = []
                for s in range(GPC):
                    col = plsc.load_gather(cur, [rowsel[s], cold])
                    nd.append(dots[s] + col * vnd)
                    nq.append(qs[s] + col * col)
                return tuple(nd), tuple(nq)

            zero7 = tuple(jnp.zeros((L,), jnp.float32) for _ in range(GPC))
            dots, qs = lax.fori_loop(0, D, dbody, (zero7, zero7), unroll=2)

            for s in range(GPC):
                q = qs[s]
                sq = q * _rsqrt16(jnp.maximum(q, 1e-30))
                r = 1.0 - dots[s] / jnp.maximum(sq, 1e-8)
                g = base + k * CK + s * L + iota
                acc = acc + jnp.where(g < M, r * r, 0.0)

        acc_v[...] = acc
        pltpu.sync_copy(acc_v, out_hbm.at[wid])

    return _sc_loss


def kernel(dst_x, enc_w1, enc_b1, enc_w2, enc_b2, e2d_w,
           dec_w1, dec_b1, dec_w2, dec_b2, mask_nodes):
    idx = jnp.zeros((M_PAD,), jnp.int32).at[:M].set(mask_nodes.astype(jnp.int32))
    partial = _make_sc_loss()(
        dst_x, idx.reshape(NW, NCK, CK), dec_b1, dec_w2, dec_b2)
    return jnp.sum(partial) / jnp.float32(M)


# speedup vs baseline: 1.9667x; 1.9667x over previous
import functools

import jax
import jax.numpy as jnp
from jax import lax
from jax.experimental import pallas as pl
from jax.experimental.pallas import tpu as pltpu
from jax.experimental.pallas import tpu_sc as plsc

N = 100000
D = 128
DH = 256
M = 50000

NC = 2
NS = 16
L = 16
NW = NC * NS
PER_W = 1664
M_PAD = NW * PER_W
CK = 128
NCK = PER_W // CK
GPC = CK // L

BLK = 16384


def _tc_dot_q(xa_ref, xb_ref, b1_ref, w2_ref, b2_ref, dot_ref, q_ref):
    v = jnp.maximum(b1_ref[...], 0.0) @ w2_ref[...] + b2_ref[...]
    vn = v / jnp.maximum(jnp.sqrt(jnp.sum(v * v)), 1e-8)
    xa = xa_ref[...]
    xb = xb_ref[...]
    contract = (((1,), (1,)), ((), ()))
    ones = jnp.ones((1, D), jnp.float32)

    def dq(x):
        dot = lax.dot_general(vn, x, contract,
                              preferred_element_type=jnp.float32)
        q = lax.dot_general(ones, x * x, contract,
                            preferred_element_type=jnp.float32)
        return dot.reshape(BLK), q.reshape(BLK)

    da, qa = dq(xa)
    db, qb = dq(xb)
    dot_ref[...] = jnp.concatenate([da, db])
    q_ref[...] = jnp.concatenate([qa, qb])


def _rsqrt16(x):
    i = plsc.bitcast(x, jnp.int32)
    y = plsc.bitcast(jnp.int32(0x5F3759DF) - (i >> 1), jnp.float32)
    for _ in range(3):
        y = y * (1.5 - 0.5 * x * y * y)
    return y


@functools.lru_cache(maxsize=1)
def _make_sc_gather_loss():
    mesh = plsc.VectorSubcoreMesh(
        core_axis_name="c", subcore_axis_name="s", num_cores=NC, num_subcores=NS
    )

    @functools.partial(
        pl.kernel,
        out_type=jax.ShapeDtypeStruct((NW, L), jnp.float32),
        mesh=mesh,
        compiler_params=pltpu.CompilerParams(needs_layout_passes=False),
        scratch_types=[
            pltpu.VMEM((NCK, CK), jnp.int32),
            pltpu.VMEM((4, CK), jnp.float32),
            pltpu.VMEM((4, CK), jnp.float32),
            pltpu.VMEM((L,), jnp.float32),
            pltpu.SemaphoreType.DMA,
            pltpu.SemaphoreType.DMA,
            pltpu.SemaphoreType.DMA,
            pltpu.SemaphoreType.DMA,
        ],
    )
    def _sc_gather_loss(dot_hbm, q_hbm, idx_hbm, out_hbm,
                        idx_v, dbuf, qbuf, acc_v, sem0, sem1, sem2, sem3):
        wid = lax.axis_index("s") * NC + lax.axis_index("c")
        base = wid * PER_W
        pltpu.sync_copy(idx_hbm.at[wid], idx_v)

        sems = (sem0, sem1, sem2, sem3)
        RB = 4

        def fire(k):
            s = sems[k % RB]
            hd = pltpu.async_copy(dot_hbm.at[idx_v.at[k]], dbuf.at[k % RB], s)
            hq = pltpu.async_copy(q_hbm.at[idx_v.at[k]], qbuf.at[k % RB], s)
            return hd, hq

        acc = jnp.zeros((L,), jnp.float32)
        pend = [fire(k) for k in range(RB - 1)]
        for k in range(NCK):
            if k + RB - 1 < NCK:
                pend.append(fire(k + RB - 1))
            hd, hq = pend.pop(0)
            hd.wait()
            hq.wait()
            for i in range(GPC):
                dot = dbuf[k % RB, pl.ds(i * L, L)]
                q = qbuf[k % RB, pl.ds(i * L, L)]
                s = q * _rsqrt16(jnp.maximum(q, 1e-30))
                r = 1.0 - dot / jnp.maximum(s, 1e-8)
                g = base + k * CK + i * L + lax.iota(jnp.int32, L)
                acc = acc + jnp.where(g < M, r * r, 0.0)

        acc_v[...] = acc
        pltpu.sync_copy(acc_v, out_hbm.at[wid])

    return _sc_gather_loss


def kernel(dst_x, enc_w1, enc_b1, enc_w2, enc_b2, e2d_w,
           dec_w1, dec_b1, dec_w2, dec_b2, mask_nodes):
    dot, q = pl.pallas_call(
        _tc_dot_q,
        grid=(pl.cdiv(N, 2 * BLK),),
        in_specs=[
            pl.BlockSpec((BLK, D), lambda i: (2 * i, 0)),
            pl.BlockSpec((BLK, D),
                         lambda i: (jnp.minimum(2 * i + 1, N // BLK), 0)),
            pl.BlockSpec((1, DH), lambda i: (0, 0)),
            pl.BlockSpec((DH, D), lambda i: (0, 0)),
            pl.BlockSpec((1, D), lambda i: (0, 0)),
        ],
        out_specs=[
            pl.BlockSpec((2 * BLK,), lambda i: (i,)),
            pl.BlockSpec((2 * BLK,), lambda i: (i,)),
        ],
        out_shape=[
            jax.ShapeDtypeStruct((N,), jnp.float32),
            jax.ShapeDtypeStruct((N,), jnp.float32),
        ],
    )(dst_x, dst_x, dec_b1.reshape(1, DH), dec_w2, dec_b2.reshape(1, D))

    idx = jnp.zeros((M_PAD,), jnp.int32).at[:M].set(mask_nodes.astype(jnp.int32))
    partial = _make_sc_gather_loss()(dot, q, idx.reshape(NW, NCK, CK))

    return jnp.sum(partial) / jnp.float32(M)


# speedup vs baseline: 2.0437x; 1.0392x over previous
import functools

import jax
import jax.numpy as jnp
from jax import lax
from jax.experimental import pallas as pl
from jax.experimental.pallas import tpu as pltpu
from jax.experimental.pallas import tpu_sc as plsc

N = 100000
D = 128
DH = 256
M = 50000

NC = 2
NS = 16
L = 16
NW = NC * NS
PER_W = 1664
M_PAD = NW * PER_W
CK = 128
NCK = PER_W // CK
GPC = CK // L

BLK = 8192


def _tc_dot_q(xa_ref, xb_ref, b1_ref, w2_ref, b2_ref, dot_ref, q_ref):
    v = jnp.maximum(b1_ref[...], 0.0) @ w2_ref[...] + b2_ref[...]
    vn = v / jnp.maximum(jnp.sqrt(jnp.sum(v * v)), 1e-8)
    xa = xa_ref[...]
    xb = xb_ref[...]
    contract = (((1,), (1,)), ((), ()))
    ones = jnp.ones((1, D), jnp.float32)

    def dq(x):
        dot = lax.dot_general(vn, x, contract,
                              preferred_element_type=jnp.float32)
        q = lax.dot_general(ones, x * x, contract,
                            preferred_element_type=jnp.float32)
        return dot.reshape(BLK), q.reshape(BLK)

    da, qa = dq(xa)
    db, qb = dq(xb)
    dot_ref[...] = jnp.concatenate([da, db])
    q_ref[...] = jnp.concatenate([qa, qb])


def _rsqrt16(x):
    i = plsc.bitcast(x, jnp.int32)
    y = plsc.bitcast(jnp.int32(0x5F3759DF) - (i >> 1), jnp.float32)
    for _ in range(3):
        y = y * (1.5 - 0.5 * x * y * y)
    return y


@functools.lru_cache(maxsize=1)
def _make_sc_gather_loss():
    mesh = plsc.VectorSubcoreMesh(
        core_axis_name="c", subcore_axis_name="s", num_cores=NC, num_subcores=NS
    )

    @functools.partial(
        pl.kernel,
        out_type=jax.ShapeDtypeStruct((NW, L), jnp.float32),
        mesh=mesh,
        compiler_params=pltpu.CompilerParams(needs_layout_passes=False),
        scratch_types=[
            pltpu.VMEM((NCK, CK), jnp.int32),
            pltpu.VMEM((4, CK), jnp.float32),
            pltpu.VMEM((4, CK), jnp.float32),
            pltpu.VMEM((L,), jnp.float32),
            pltpu.SemaphoreType.DMA,
            pltpu.SemaphoreType.DMA,
            pltpu.SemaphoreType.DMA,
            pltpu.SemaphoreType.DMA,
        ],
    )
    def _sc_gather_loss(dot_hbm, q_hbm, idx_hbm, out_hbm,
                        idx_v, dbuf, qbuf, acc_v, sem0, sem1, sem2, sem3):
        wid = lax.axis_index("s") * NC + lax.axis_index("c")
        base = wid * PER_W
        pltpu.sync_copy(idx_hbm.at[wid], idx_v)

        sems = (sem0, sem1, sem2, sem3)
        RB = 4

        def fire(k):
            s = sems[k % RB]
            hd = pltpu.async_copy(dot_hbm.at[idx_v.at[k]], dbuf.at[k % RB], s)
            hq = pltpu.async_copy(q_hbm.at[idx_v.at[k]], qbuf.at[k % RB], s)
            return hd, hq

        acc = jnp.zeros((L,), jnp.float32)
        pend = [fire(k) for k in range(RB - 1)]
        for k in range(NCK):
            if k + RB - 1 < NCK:
                pend.append(fire(k + RB - 1))
            hd, hq = pend.pop(0)
            hd.wait()
            hq.wait()
            for i in range(GPC):
                dot = dbuf[k % RB, pl.ds(i * L, L)]
                q = qbuf[k % RB, pl.ds(i * L, L)]
                s = q * _rsqrt16(jnp.maximum(q, 1e-30))
                r = 1.0 - dot / jnp.maximum(s, 1e-8)
                g = base + k * CK + i * L + lax.iota(jnp.int32, L)
                acc = acc + jnp.where(g < M, r * r, 0.0)

        acc_v[...] = acc
        pltpu.sync_copy(acc_v, out_hbm.at[wid])

    return _sc_gather_loss


def kernel(dst_x, enc_w1, enc_b1, enc_w2, enc_b2, e2d_w,
           dec_w1, dec_b1, dec_w2, dec_b2, mask_nodes):
    dot, q = pl.pallas_call(
        _tc_dot_q,
        grid=(pl.cdiv(N, 2 * BLK),),
        in_specs=[
            pl.BlockSpec((BLK, D), lambda i: (2 * i, 0)),
            pl.BlockSpec((BLK, D),
                         lambda i: (jnp.minimum(2 * i + 1, N // BLK), 0)),
            pl.BlockSpec((1, DH), lambda i: (0, 0)),
            pl.BlockSpec((DH, D), lambda i: (0, 0)),
            pl.BlockSpec((1, D), lambda i: (0, 0)),
        ],
        out_specs=[
            pl.BlockSpec((2 * BLK,), lambda i: (i,)),
            pl.BlockSpec((2 * BLK,), lambda i: (i,)),
        ],
        out_shape=[
            jax.ShapeDtypeStruct((N,), jnp.float32),
            jax.ShapeDtypeStruct((N,), jnp.float32),
        ],
    )(dst_x, dst_x, dec_b1.reshape(1, DH), dec_w2, dec_b2.reshape(1, D))

    idx = jnp.zeros((M_PAD,), jnp.int32).at[:M].set(mask_nodes.astype(jnp.int32))
    partial = _make_sc_gather_loss()(dot, q, idx.reshape(NW, NCK, CK))

    return jnp.sum(partial) / jnp.float32(M)


# speedup vs baseline: 2.6786x; 1.3107x over previous
import functools

import jax
import jax.numpy as jnp
from jax import lax
from jax.experimental import pallas as pl
from jax.experimental.pallas import tpu as pltpu
from jax.experimental.pallas import tpu_sc as plsc

N = 100000
D = 128
DH = 256
M = 50000

NC = 2
NS = 16
L = 16
NW = NC * NS
PER_W = 1568
M_PAD = NW * PER_W
CK = 112
NCK = PER_W // CK
GPC = CK // L

BLK = 8192


def _tc_dot_q(xa_ref, xb_ref, b1_ref, w2_ref, b2_ref, dot_ref, q_ref):
    v = jnp.maximum(b1_ref[...], 0.0) @ w2_ref[...] + b2_ref[...]
    vn = v / jnp.maximum(jnp.sqrt(jnp.sum(v * v)), 1e-8)
    xa = xa_ref[...]
    xb = xb_ref[...]
    contract = (((1,), (1,)), ((), ()))
    ones = jnp.ones((1, D), jnp.float32)

    def dq(x):
        dot = lax.dot_general(vn, x, contract,
                              preferred_element_type=jnp.float32)
        q = lax.dot_general(ones, x * x, contract,
                            preferred_element_type=jnp.float32)
        return dot.reshape(BLK), q.reshape(BLK)

    da, qa = dq(xa)
    db, qb = dq(xb)
    dot_ref[...] = jnp.concatenate([da, db])
    q_ref[...] = jnp.concatenate([qa, qb])


def _rsqrt16(x):
    i = plsc.bitcast(x, jnp.int32)
    y = plsc.bitcast(jnp.int32(0x5F3759DF) - (i >> 1), jnp.float32)
    for _ in range(3):
        y = y * (1.5 - 0.5 * x * y * y)
    return y


@functools.lru_cache(maxsize=1)
def _make_sc_gather_loss():
    mesh = plsc.VectorSubcoreMesh(
        core_axis_name="c", subcore_axis_name="s", num_cores=NC, num_subcores=NS
    )

    @functools.partial(
        pl.kernel,
        out_type=jax.ShapeDtypeStruct((NW, L), jnp.float32),
        mesh=mesh,
        compiler_params=pltpu.CompilerParams(needs_layout_passes=False),
        scratch_types=[
            pltpu.VMEM((NCK, CK), jnp.int32),
            pltpu.VMEM((4, CK), jnp.float32),
            pltpu.VMEM((4, CK), jnp.float32),
            pltpu.VMEM((L,), jnp.float32),
            pltpu.SemaphoreType.DMA,
            pltpu.SemaphoreType.DMA,
            pltpu.SemaphoreType.DMA,
            pltpu.SemaphoreType.DMA,
        ],
    )
    def _sc_gather_loss(dot_hbm, q_hbm, idx_hbm, out_hbm,
                        idx_v, dbuf, qbuf, acc_v, sem0, sem1, sem2, sem3):
        wid = lax.axis_index("s") * NC + lax.axis_index("c")
        base = wid * PER_W
        pltpu.sync_copy(idx_hbm.at[wid], idx_v)

        sems = (sem0, sem1, sem2, sem3)
        RB = 4

        def fire(k):
            s = sems[k % RB]
            hd = pltpu.async_copy(dot_hbm.at[idx_v.at[k]], dbuf.at[k % RB], s)
            hq = pltpu.async_copy(q_hbm.at[idx_v.at[k]], qbuf.at[k % RB], s)
            return hd, hq

        acc = jnp.zeros((L,), jnp.float32)
        pend = [fire(k) for k in range(RB - 1)]
        for k in range(NCK):
            if k + RB - 1 < NCK:
                pend.append(fire(k + RB - 1))
            hd, hq = pend.pop(0)
            hd.wait()
            hq.wait()
            for i in range(GPC):
                dot = dbuf[k % RB, pl.ds(i * L, L)]
                q = qbuf[k % RB, pl.ds(i * L, L)]
                s = q * _rsqrt16(jnp.maximum(q, 1e-30))
                r = 1.0 - dot / jnp.maximum(s, 1e-8)
                g = base + k * CK + i * L + lax.iota(jnp.int32, L)
                acc = acc + jnp.where(g < M, r * r, 0.0)

        acc_v[...] = acc
        pltpu.sync_copy(acc_v, out_hbm.at[wid])

    return _sc_gather_loss


def kernel(dst_x, enc_w1, enc_b1, enc_w2, enc_b2, e2d_w,
           dec_w1, dec_b1, dec_w2, dec_b2, mask_nodes):
    dot, q = pl.pallas_call(
        _tc_dot_q,
        grid=(pl.cdiv(N, 2 * BLK),),
        in_specs=[
            pl.BlockSpec((BLK, D), lambda i: (2 * i, 0)),
            pl.BlockSpec((BLK, D),
                         lambda i: (jnp.minimum(2 * i + 1, N // BLK), 0)),
            pl.BlockSpec((1, DH), lambda i: (0, 0)),
            pl.BlockSpec((DH, D), lambda i: (0, 0)),
            pl.BlockSpec((1, D), lambda i: (0, 0)),
        ],
        out_specs=[
            pl.BlockSpec((2 * BLK,), lambda i: (i,)),
            pl.BlockSpec((2 * BLK,), lambda i: (i,)),
        ],
        out_shape=[
            jax.ShapeDtypeStruct((N,), jnp.float32),
            jax.ShapeDtypeStruct((N,), jnp.float32),
        ],
    )(dst_x, dst_x, dec_b1.reshape(1, DH), dec_w2, dec_b2.reshape(1, D))

    idx = jnp.zeros((M_PAD,), jnp.int32).at[:M].set(mask_nodes.astype(jnp.int32))
    partial = _make_sc_gather_loss()(dot, q, idx.reshape(NW, NCK, CK))

    return jnp.sum(partial) / jnp.float32(M)
